# baseline (speedup 1.0000x reference)
import jax
import jax.numpy as jnp
from jax import lax
from jax.experimental import pallas as pl
from jax.experimental.pallas import tpu as pltpu
from jax.experimental.pallas import tpu_sc as plsc

NC = 2
NS = 16
NW = NC * NS
CH = 40
G = 5


def _sc_mesh():
    return plsc.VectorSubcoreMesh(core_axis_name="c", subcore_axis_name="s")


def _gather_call(nfeat, idx3d, E, D):
    n_chunks = E // CH
    cpt = n_chunks // NW
    ngr = cpt // G

    def body(nfeat_hbm, idx_hbm, gs_hbm, gd_hbm, idx_v, big0, big1, rs0, rs1,
             ws0, ws1):
        wid = lax.axis_index("s") * NC + lax.axis_index("c")

        for a, out in ((0, gs_hbm), (1, gd_hbm)):
            pltpu.sync_copy(idx_hbm.at[a * NW + wid], idx_v)
            base = wid * cpt * CH

            def fire(g, buf, sem):
                for k in range(G):
                    pltpu.async_copy(nfeat_hbm.at[idx_v.at[g * G + k]],
                                     buf.at[pl.ds(k * CH, CH), :], sem)

            def drain_reads(buf, sem):
                for k in range(G):
                    pltpu.make_async_copy(
                        nfeat_hbm.at[idx_v.at[k]],
                        buf.at[pl.ds(k * CH, CH), :], sem).wait()

            def write(g, buf, sem):
                return pltpu.async_copy(
                    buf, out.at[pl.ds(base + g * G * CH, G * CH), :], sem)

            fire(0, big0, rs0)

            @pl.loop(0, ngr - 1, step=2)
            def _(g):
                fire(g + 1, big1, rs1)
                drain_reads(big0, rs0)
                write(g, big0, ws0)
                pltpu.make_async_copy(
                    big0, out.at[pl.ds(base, G * CH), :], ws0).wait()
                fire(g + 2, big0, rs0)
                drain_reads(big1, rs1)
                write(g + 1, big1, ws1)
                pltpu.make_async_copy(
                    big1, out.at[pl.ds(base, G * CH), :], ws1).wait()

            drain_reads(big0, rs0)
            pltpu.sync_copy(
                big0, out.at[pl.ds(base + (ngr - 1) * G * CH, G * CH), :])

    f = pl.kernel(
        body,
        out_type=(jax.ShapeDtypeStruct((E, D), jnp.float32),
                  jax.ShapeDtypeStruct((E, D), jnp.float32)),
        mesh=_sc_mesh(),
        scratch_types=[
            pltpu.VMEM((cpt, CH), jnp.int32),
            pltpu.VMEM((G * CH, D), jnp.float32),
            pltpu.VMEM((G * CH, D), jnp.float32),
            pltpu.SemaphoreType.DMA,
            pltpu.SemaphoreType.DMA,
            pltpu.SemaphoreType.DMA,
            pltpu.SemaphoreType.DMA,
        ],
    )
    return f(nfeat, idx3d)


def _scatter_call(efeat, didx3d, zeros_nd, N, E, D):
    n_chunks = E // CH
    cpt = n_chunks // NW
    ZR = 1000

    def body(efeat_hbm, idx_hbm, zeros_hbm, parts_hbm, idx_v, big0, big1,
             table, r0, r1, a0, a1):
        c = lax.axis_index("c")
        s = lax.axis_index("s")
        wid = c * NS + s

        @pl.when(s < N // ZR)
        def _():
            pltpu.sync_copy(zeros_hbm.at[pl.ds(s * ZR, ZR), :],
                            table.at[pl.ds(s * ZR, ZR), :])

        plsc.subcore_barrier()
        pltpu.sync_copy(idx_hbm.at[wid], idx_v)
        base = wid * cpt * CH
        W = 3

        def fire_reads(g, buf, rsem):
            for k in range(W):
                pltpu.async_copy(
                    efeat_hbm.at[pl.ds(base + (g * W + k) * CH, CH), :],
                    buf.at[pl.ds(k * CH, CH), :], rsem)

        def drain_reads(buf, rsem):
            for k in range(W):
                pltpu.make_async_copy(
                    efeat_hbm.at[pl.ds(base, CH), :],
                    buf.at[pl.ds(k * CH, CH), :], rsem).wait()

        def adds(g, buf, asem):
            for k in range(W):
                pltpu.async_copy(buf.at[pl.ds(k * CH, CH), :],
                                 table.at[idx_v.at[g * W + k]], asem,
                                 add=True)
            for k in range(W):
                pltpu.make_async_copy(buf.at[pl.ds(k * CH, CH), :],
                                      table.at[idx_v.at[k]], asem).wait()

        ngr = cpt // W

        fire_reads(0, big0, r0)

        @pl.loop(0, ngr - 1, step=2)
        def _(g):
            fire_reads(g + 1, big1, r1)
            drain_reads(big0, r0)
            adds(g, big0, a0)
            fire_reads(g + 2, big0, r0)
            drain_reads(big1, r1)
            adds(g + 1, big1, a1)

        drain_reads(big0, r0)
        adds(ngr - 1, big0, a0)
        for j in range(ngr * W, cpt):
            pltpu.sync_copy(efeat_hbm.at[pl.ds(base + j * CH, CH), :],
                            big1.at[pl.ds(0, CH), :])
            pltpu.sync_copy(big1.at[pl.ds(0, CH), :],
                            table.at[idx_v.at[j]], add=True)

        plsc.subcore_barrier()

        @pl.when(s < N // ZR)
        def _():
            pltpu.sync_copy(table.at[pl.ds(s * ZR, ZR), :],
                            parts_hbm.at[pl.ds(c * N + s * ZR, ZR), :])

    f = pl.kernel(
        body,
        out_type=jax.ShapeDtypeStruct((2 * N, D), jnp.float32),
        mesh=_sc_mesh(),
        scratch_types=[
            pltpu.VMEM((cpt, CH), jnp.int32),
            pltpu.VMEM((3 * CH, D), jnp.float32),
            pltpu.VMEM((3 * CH, D), jnp.float32),
            pltpu.VMEM_SHARED((N, D), jnp.float32),
            pltpu.SemaphoreType.DMA,
            pltpu.SemaphoreType.DMA,
            pltpu.SemaphoreType.DMA,
            pltpu.SemaphoreType.DMA,
        ],
    )
    return f(efeat, didx3d, zeros_nd)


def _bdot(x16, w_ref, b_ref, relu, out_f32=False):
    y = jnp.dot(x16, w_ref[...].astype(jnp.bfloat16),
                preferred_element_type=jnp.float32) + b_ref[...]
    if relu:
        y = jnp.maximum(y, 0.0)
    return y if out_f32 else y.astype(jnp.bfloat16)


def _layer_norm(y, g, b):
    m = jnp.mean(y, axis=-1, keepdims=True)
    v = jnp.mean((y - m) ** 2, axis=-1, keepdims=True)
    return (y - m) * lax.rsqrt(v + 1e-5) * g + b


def _edge_mlp_kernel(e_ref, gs_ref, gd_ref, w1_ref, b1_ref, w2_ref, b2_ref,
                     w3_ref, b3_ref, g_ref, beta_ref, out_ref):
    e = e_ref[...]
    x = jnp.concatenate([e, gs_ref[...], gd_ref[...]], axis=1)
    h = _bdot(x.astype(jnp.bfloat16), w1_ref[...], b1_ref[...], relu=True)
    h = _bdot(h, w2_ref[...], b2_ref[...], relu=True)
    y = _bdot(h, w3_ref[...], b3_ref[...], relu=False, out_f32=True)
    out_ref[...] = _layer_norm(y, g_ref[...], beta_ref[...]) + e


def _edge_mlp_call(efeat, gs, gd, w1, b1, w2, b2, w3, b3, g, beta, E, D):
    BR = 2000
    grid = (E // BR,)
    row = lambda i: (i, 0)
    full = lambda i: (0, 0)
    return pl.pallas_call(
        _edge_mlp_kernel,
        grid=grid,
        in_specs=[
            pl.BlockSpec((BR, D), row),
            pl.BlockSpec((BR, D), row),
            pl.BlockSpec((BR, D), row),
            pl.BlockSpec((3 * D, D), full),
            pl.BlockSpec((1, D), full),
            pl.BlockSpec((D, D), full),
            pl.BlockSpec((1, D), full),
            pl.BlockSpec((D, D), full),
            pl.BlockSpec((1, D), full),
            pl.BlockSpec((1, D), full),
            pl.BlockSpec((1, D), full),
        ],
        out_specs=pl.BlockSpec((BR, D), row),
        out_shape=jax.ShapeDtypeStruct((E, D), jnp.float32),
    )(efeat, gs, gd, w1, b1, w2, b2, w3, b3, g, beta)


def _node_mlp_kernel(p_ref, nf_ref, w1_ref, b1_ref, w2_ref, b2_ref, w3_ref,
                     b3_ref, g_ref, beta_ref, out_ref):
    nf = nf_ref[...]
    agg = p_ref[0] + p_ref[1]
    x = jnp.concatenate([agg, nf], axis=1)
    h = _bdot(x.astype(jnp.bfloat16), w1_ref[...], b1_ref[...], relu=True)
    h = _bdot(h, w2_ref[...], b2_ref[...], relu=True)
    y = _bdot(h, w3_ref[...], b3_ref[...], relu=False, out_f32=True)
    out_ref[...] = _layer_norm(y, g_ref[...], beta_ref[...]) + nf


def _node_mlp_call(parts, nfeat, w1, b1, w2, b2, w3, b3, g, beta, N, D):
    BR = 2000
    grid = (N // BR,)
    row = lambda i: (i, 0)
    full = lambda i: (0, 0)
    parts3 = parts.reshape(2, N, D)
    return pl.pallas_call(
        _node_mlp_kernel,
        grid=grid,
        in_specs=[
            pl.BlockSpec((2, BR, D), lambda i: (0, i, 0)),
            pl.BlockSpec((BR, D), row),
            pl.BlockSpec((2 * D, D), full),
            pl.BlockSpec((1, D), full),
            pl.BlockSpec((D, D), full),
            pl.BlockSpec((1, D), full),
            pl.BlockSpec((D, D), full),
            pl.BlockSpec((1, D), full),
            pl.BlockSpec((1, D), full),
            pl.BlockSpec((1, D), full),
        ],
        out_specs=pl.BlockSpec((BR, D), row),
        out_shape=jax.ShapeDtypeStruct((N, D), jnp.float32),
    )(parts3, nfeat, w1, b1, w2, b2, w3, b3, g, beta)


def kernel(node_features, edge_features, edge_index, context_node, context_edge,
           eW1, eb1, eW2, eb2, eW3, eb3, eg, ebeta,
           nW1, nb1, nW2, nb2, nW3, nb3, ng, nbeta):
    N, D = node_features.shape
    E = edge_features.shape[0]
    L = eW1.shape[0]
    cpt = E // CH // NW

    idx3d = edge_index.reshape(2 * NW, cpt, CH)
    didx3d = edge_index[1].reshape(NW, cpt, CH)
    zeros_nd = jnp.zeros((N, D), jnp.float32)

    r = lambda b: b.reshape(1, D)

    nfeat = node_features
    efeat = edge_features
    for l in range(L):
        gs, gd = _gather_call(nfeat, idx3d, E, D)
        efeat = _edge_mlp_call(efeat, gs, gd, eW1[l], r(eb1[l]), eW2[l],
                               r(eb2[l]), eW3[l], r(eb3[l]), r(eg[l]),
                               r(ebeta[l]), E, D)
        parts = _scatter_call(efeat, didx3d, zeros_nd, N, E, D)
        nfeat = _node_mlp_call(parts, nfeat, nW1[l], r(nb1[l]), nW2[l],
                               r(nb2[l]), nW3[l], r(nb3[l]), r(ng[l]),
                               r(nbeta[l]), N, D)
    return nfeat


# speedup vs baseline: 1.0570x; 1.0570x over previous
import jax
import jax.numpy as jnp
from jax import lax
from jax.experimental import pallas as pl
from jax.experimental.pallas import tpu as pltpu
from jax.experimental.pallas import tpu_sc as plsc

NC = 2
NS = 16
NW = NC * NS
CH = 40
G = 5


def _pingpong(nitems, fire, consume):
    if nitems <= 0:
        return
    fire(0, 0)

    @pl.loop(0, (nitems + 1) // 2)
    def _(p):
        i0 = 2 * p

        @pl.when(i0 + 1 < nitems)
        def _():
            fire(i0 + 1, 1)

        consume(i0, 0)

        @pl.when(i0 + 2 < nitems)
        def _():
            fire(i0 + 2, 0)

        @pl.when(i0 + 1 < nitems)
        def _():
            consume(i0 + 1, 1)


def _sc_mesh():
    return plsc.VectorSubcoreMesh(core_axis_name="c", subcore_axis_name="s")


def _gather_call(nfeat, idx3d, E, D):
    n_chunks = E // CH
    cpt = n_chunks // NW
    nfull = cpt // G
    rem = cpt - nfull * G

    def body(nfeat_hbm, idx_hbm, gs_hbm, gd_hbm, idx_v, big0, big1, rs0, rs1,
             ws0, ws1):
        wid = lax.axis_index("s") * NC + lax.axis_index("c")
        bufs = (big0, big1)
        rsems = (rs0, rs1)
        wsems = (ws0, ws1)

        for a, out in ((0, gs_hbm), (1, gd_hbm)):
            pltpu.sync_copy(idx_hbm.at[a * NW + wid], idx_v)
            base = wid * cpt * CH

            def fire(g, slot):
                for k in range(G):
                    pltpu.async_copy(nfeat_hbm.at[idx_v.at[g * G + k]],
                                     bufs[slot].at[pl.ds(k * CH, CH), :],
                                     rsems[slot])

            def consume(g, slot):
                buf = bufs[slot]
                for k in range(G):
                    pltpu.make_async_copy(
                        nfeat_hbm.at[idx_v.at[k]],
                        buf.at[pl.ds(k * CH, CH), :], rsems[slot]).wait()
                dst = out.at[pl.ds(base + g * G * CH, G * CH), :]
                pltpu.async_copy(buf, dst, wsems[slot])
                pltpu.make_async_copy(buf, dst, wsems[slot]).wait()

            _pingpong(nfull, fire, consume)
            for t in range(rem):
                j = nfull * G + t
                pltpu.async_copy(nfeat_hbm.at[idx_v.at[j]],
                                 big0.at[pl.ds(0, CH), :], rs0).wait()
                pltpu.sync_copy(big0.at[pl.ds(0, CH), :],
                                out.at[pl.ds(base + j * CH, CH), :])

    f = pl.kernel(
        body,
        out_type=(jax.ShapeDtypeStruct((E, D), jnp.float32),
                  jax.ShapeDtypeStruct((E, D), jnp.float32)),
        mesh=_sc_mesh(),
        scratch_types=[
            pltpu.VMEM((cpt, CH), jnp.int32),
            pltpu.VMEM((G * CH, D), jnp.float32),
            pltpu.VMEM((G * CH, D), jnp.float32),
            pltpu.SemaphoreType.DMA,
            pltpu.SemaphoreType.DMA,
            pltpu.SemaphoreType.DMA,
            pltpu.SemaphoreType.DMA,
        ],
    )
    return f(nfeat, idx3d)


def _scatter_call(efA, efB, idxA3d, idxB3d, zeros_nd, N, D):
    cptA = efA.shape[0] // CH // NW
    cptB = efB.shape[0] // CH // NW
    ZR = 1000
    W = 3

    def body(efA_hbm, efB_hbm, idxA_hbm, idxB_hbm, zeros_hbm, parts_hbm,
             idx_v, big0, big1, table, r0, r1, a0, a1):
        c = lax.axis_index("c")
        s = lax.axis_index("s")
        wid = c * NS + s

        @pl.when(s < N // ZR)
        def _():
            pltpu.sync_copy(zeros_hbm.at[pl.ds(s * ZR, ZR), :],
                            table.at[pl.ds(s * ZR, ZR), :])

        plsc.subcore_barrier()
        bufs = (big0, big1)
        rsems = (r0, r1)
        asems = (a0, a1)

        for ef_hbm, idx_hbm, cpt in ((efA_hbm, idxA_hbm, cptA),
                                     (efB_hbm, idxB_hbm, cptB)):
            pltpu.sync_copy(idx_hbm.at[wid],
                            idx_v.at[pl.ds(0, cpt), :])
            base = wid * cpt * CH
            nfull = cpt // W
            rem = cpt - nfull * W

            def fire(g, slot):
                for k in range(W):
                    pltpu.async_copy(
                        ef_hbm.at[pl.ds(base + (g * W + k) * CH, CH), :],
                        bufs[slot].at[pl.ds(k * CH, CH), :], rsems[slot])

            def consume(g, slot):
                buf = bufs[slot]
                for k in range(W):
                    pltpu.make_async_copy(
                        ef_hbm.at[pl.ds(base, CH), :],
                        buf.at[pl.ds(k * CH, CH), :], rsems[slot]).wait()
                for k in range(W):
                    pltpu.async_copy(buf.at[pl.ds(k * CH, CH), :],
                                     table.at[idx_v.at[g * W + k]],
                                     asems[slot], add=True)
                for k in range(W):
                    pltpu.make_async_copy(buf.at[pl.ds(k * CH, CH), :],
                                          table.at[idx_v.at[k]],
                                          asems[slot]).wait()

            _pingpong(nfull, fire, consume)
            for t in range(rem):
                j = nfull * W + t
                pltpu.sync_copy(ef_hbm.at[pl.ds(base + j * CH, CH), :],
                                big1.at[pl.ds(0, CH), :])
                pltpu.sync_copy(big1.at[pl.ds(0, CH), :],
                                table.at[idx_v.at[j]], add=True)

        plsc.subcore_barrier()

        @pl.when(s < N // ZR)
        def _():
            pltpu.sync_copy(table.at[pl.ds(s * ZR, ZR), :],
                            parts_hbm.at[pl.ds(c * N + s * ZR, ZR), :])

    cpt_max = max(cptA, cptB)
    f = pl.kernel(
        body,
        out_type=jax.ShapeDtypeStruct((2 * N, D), jnp.float32),
        mesh=_sc_mesh(),
        scratch_types=[
            pltpu.VMEM((cpt_max, CH), jnp.int32),
            pltpu.VMEM((W * CH, D), jnp.float32),
            pltpu.VMEM((W * CH, D), jnp.float32),
            pltpu.VMEM_SHARED((N, D), jnp.float32),
            pltpu.SemaphoreType.DMA,
            pltpu.SemaphoreType.DMA,
            pltpu.SemaphoreType.DMA,
            pltpu.SemaphoreType.DMA,
        ],
    )
    return f(efA, efB, idxA3d, idxB3d, zeros_nd)


def _layer_norm(y, g, b):
    m = jnp.mean(y, axis=-1, keepdims=True)
    v = jnp.mean((y - m) ** 2, axis=-1, keepdims=True)
    return (y - m) * lax.rsqrt(v + 1e-5) * g + b


def _edge_mlp_kernel(e_ref, gs_ref, gd_ref, w1_ref, b1_ref, w2_ref, b2_ref,
                     w3_ref, b3_ref, g_ref, beta_ref, out_ref):
    e = e_ref[...]
    x = jnp.concatenate([e, gs_ref[...], gd_ref[...]], axis=1)
    h = jnp.maximum(
        jnp.dot(x, w1_ref[...], preferred_element_type=jnp.float32)
        + b1_ref[...], 0.0)
    h = jnp.maximum(
        jnp.dot(h, w2_ref[...], preferred_element_type=jnp.float32)
        + b2_ref[...], 0.0)
    y = jnp.dot(h, w3_ref[...], preferred_element_type=jnp.float32) + b3_ref[...]
    out_ref[...] = _layer_norm(y, g_ref[...], beta_ref[...]) + e


def _edge_mlp_call(efeat, gs, gd, w1, b1, w2, b2, w3, b3, g, beta, E, D):
    BR = next(b for b in (2000, 1600, 1000, 800, 400, 200, 80, 40, 8)
              if E % b == 0)
    grid = (E // BR,)
    row = lambda i: (i, 0)
    full = lambda i: (0, 0)
    return pl.pallas_call(
        _edge_mlp_kernel,
        grid=grid,
        in_specs=[
            pl.BlockSpec((BR, D), row),
            pl.BlockSpec((BR, D), row),
            pl.BlockSpec((BR, D), row),
            pl.BlockSpec((3 * D, D), full),
            pl.BlockSpec((1, D), full),
            pl.BlockSpec((D, D), full),
            pl.BlockSpec((1, D), full),
            pl.BlockSpec((D, D), full),
            pl.BlockSpec((1, D), full),
            pl.BlockSpec((1, D), full),
            pl.BlockSpec((1, D), full),
        ],
        out_specs=pl.BlockSpec((BR, D), row),
        out_shape=jax.ShapeDtypeStruct((E, D), jnp.float32),
    )(efeat, gs, gd, w1, b1, w2, b2, w3, b3, g, beta)


def _node_mlp_kernel(p_ref, nf_ref, w1_ref, b1_ref, w2_ref, b2_ref, w3_ref,
                     b3_ref, g_ref, beta_ref, out_ref):
    nf = nf_ref[...]
    agg = p_ref[0] + p_ref[1]
    x = jnp.concatenate([agg, nf], axis=1)
    h = jnp.maximum(
        jnp.dot(x, w1_ref[...], preferred_element_type=jnp.float32)
        + b1_ref[...], 0.0)
    h = jnp.maximum(
        jnp.dot(h, w2_ref[...], preferred_element_type=jnp.float32)
        + b2_ref[...], 0.0)
    y = jnp.dot(h, w3_ref[...], preferred_element_type=jnp.float32) + b3_ref[...]
    out_ref[...] = _layer_norm(y, g_ref[...], beta_ref[...]) + nf


def _node_mlp_call(parts, nfeat, w1, b1, w2, b2, w3, b3, g, beta, N, D):
    BR = 2000
    grid = (N // BR,)
    row = lambda i: (i, 0)
    full = lambda i: (0, 0)
    parts3 = parts.reshape(2, N, D)
    return pl.pallas_call(
        _node_mlp_kernel,
        grid=grid,
        in_specs=[
            pl.BlockSpec((2, BR, D), lambda i: (0, i, 0)),
            pl.BlockSpec((BR, D), row),
            pl.BlockSpec((2 * D, D), full),
            pl.BlockSpec((1, D), full),
            pl.BlockSpec((D, D), full),
            pl.BlockSpec((1, D), full),
            pl.BlockSpec((D, D), full),
            pl.BlockSpec((1, D), full),
            pl.BlockSpec((1, D), full),
            pl.BlockSpec((1, D), full),
        ],
        out_specs=pl.BlockSpec((BR, D), row),
        out_shape=jax.ShapeDtypeStruct((N, D), jnp.float32),
    )(parts3, nfeat, w1, b1, w2, b2, w3, b3, g, beta)


def kernel(node_features, edge_features, edge_index, context_node, context_edge,
           eW1, eb1, eW2, eb2, eW3, eb3, eg, ebeta,
           nW1, nb1, nW2, nb2, nW3, nb3, ng, nbeta):
    N, D = node_features.shape
    E = edge_features.shape[0]
    L = eW1.shape[0]

    EA = (E * 2 // 5 // (CH * NW)) * (CH * NW)
    EB = E - EA
    cptA = EA // CH // NW
    cptB = EB // CH // NW

    giA = edge_index[:, :EA].reshape(2 * NW, cptA, CH)
    giB = edge_index[:, EA:].reshape(2 * NW, cptB, CH)
    diA = edge_index[1, :EA].reshape(NW, cptA, CH)
    diB = edge_index[1, EA:].reshape(NW, cptB, CH)
    zeros_nd = jnp.zeros((N, D), jnp.float32)

    r = lambda b: b.reshape(1, D)

    nfeat = node_features
    efA = edge_features[:EA]
    efB = edge_features[EA:]
    for l in range(L):
        gsA, gdA = _gather_call(nfeat, giA, EA, D)
        gsB, gdB = _gather_call(nfeat, giB, EB, D)
        efA = _edge_mlp_call(efA, gsA, gdA, eW1[l], r(eb1[l]), eW2[l],
                             r(eb2[l]), eW3[l], r(eb3[l]), r(eg[l]),
                             r(ebeta[l]), EA, D)
        efB = _edge_mlp_call(efB, gsB, gdB, eW1[l], r(eb1[l]), eW2[l],
                             r(eb2[l]), eW3[l], r(eb3[l]), r(eg[l]),
                             r(ebeta[l]), EB, D)
        parts = _scatter_call(efA, efB, diA, diB, zeros_nd, N, D)
        nfeat = _node_mlp_call(parts, nfeat, nW1[l], r(nb1[l]), nW2[l],
                               r(nb2[l]), nW3[l], r(nb3[l]), r(ng[l]),
                               r(nbeta[l]), N, D)
    return nfeat


# speedup vs baseline: 1.0999x; 1.0406x over previous
import jax
import jax.numpy as jnp
from jax import lax
from jax.experimental import pallas as pl
from jax.experimental.pallas import tpu as pltpu
from jax.experimental.pallas import tpu_sc as plsc

NC = 2
NS = 16
NW = NC * NS
CH = 40
G = 5


def _pingpong(nitems, fire, consume):
    if nitems <= 0:
        return
    fire(0, 0)

    @pl.loop(0, (nitems + 1) // 2)
    def _(p):
        i0 = 2 * p

        @pl.when(i0 + 1 < nitems)
        def _():
            fire(i0 + 1, 1)

        consume(i0, 0)

        @pl.when(i0 + 2 < nitems)
        def _():
            fire(i0 + 2, 0)

        @pl.when(i0 + 1 < nitems)
        def _():
            consume(i0 + 1, 1)


def _sc_mesh():
    return plsc.VectorSubcoreMesh(core_axis_name="c", subcore_axis_name="s")


def _gather_call(nfeat, idx3d, E, D):
    n_chunks = E // CH
    cpt = n_chunks // NW
    nfull = cpt // G
    rem = cpt - nfull * G

    def body(nfeat_hbm, idx_hbm, gs_hbm, gd_hbm, idx_v, big0, big1, rs0, rs1,
             ws0, ws1):
        wid = lax.axis_index("s") * NC + lax.axis_index("c")
        bufs = (big0, big1)
        rsems = (rs0, rs1)
        wsems = (ws0, ws1)

        for a, out in ((0, gs_hbm), (1, gd_hbm)):
            pltpu.sync_copy(idx_hbm.at[a * NW + wid], idx_v)
            base = wid * cpt * CH

            def fire(g, slot):
                for k in range(G):
                    pltpu.async_copy(nfeat_hbm.at[idx_v.at[g * G + k]],
                                     bufs[slot].at[pl.ds(k * CH, CH), :],
                                     rsems[slot])

            def consume(g, slot):
                buf = bufs[slot]
                for k in range(G):
                    pltpu.make_async_copy(
                        nfeat_hbm.at[idx_v.at[k]],
                        buf.at[pl.ds(k * CH, CH), :], rsems[slot]).wait()
                dst = out.at[pl.ds(base + g * G * CH, G * CH), :]
                pltpu.async_copy(buf, dst, wsems[slot])
                pltpu.make_async_copy(buf, dst, wsems[slot]).wait()

            _pingpong(nfull, fire, consume)
            for t in range(rem):
                j = nfull * G + t
                pltpu.async_copy(nfeat_hbm.at[idx_v.at[j]],
                                 big0.at[pl.ds(0, CH), :], rs0).wait()
                pltpu.sync_copy(big0.at[pl.ds(0, CH), :],
                                out.at[pl.ds(base + j * CH, CH), :])

    f = pl.kernel(
        body,
        out_type=(jax.ShapeDtypeStruct((E, D), jnp.float32),
                  jax.ShapeDtypeStruct((E, D), jnp.float32)),
        mesh=_sc_mesh(),
        scratch_types=[
            pltpu.VMEM((cpt, CH), jnp.int32),
            pltpu.VMEM((G * CH, D), jnp.float32),
            pltpu.VMEM((G * CH, D), jnp.float32),
            pltpu.SemaphoreType.DMA,
            pltpu.SemaphoreType.DMA,
            pltpu.SemaphoreType.DMA,
            pltpu.SemaphoreType.DMA,
        ],
    )
    return f(nfeat, idx3d)


def _scatter_call(ef, idx3d, init2n, N, D):
    cpt = ef.shape[0] // CH // NW
    ZR = 1000
    W = 3

    def body(ef_hbm, idx_hbm, init_hbm, parts_hbm, idx_v, big0, big1, table,
             r0, r1, a0, a1):
        c = lax.axis_index("c")
        s = lax.axis_index("s")
        wid = c * NS + s

        @pl.when(s < N // ZR)
        def _():
            pltpu.sync_copy(init_hbm.at[pl.ds(c * N + s * ZR, ZR), :],
                            table.at[pl.ds(s * ZR, ZR), :])

        plsc.subcore_barrier()
        bufs = (big0, big1)
        rsems = (r0, r1)
        asems = (a0, a1)

        pltpu.sync_copy(idx_hbm.at[wid], idx_v)
        base = wid * cpt * CH
        nfull = cpt // W
        rem = cpt - nfull * W

        def fire(g, slot):
            for k in range(W):
                pltpu.async_copy(
                    ef_hbm.at[pl.ds(base + (g * W + k) * CH, CH), :],
                    bufs[slot].at[pl.ds(k * CH, CH), :], rsems[slot])

        def consume(g, slot):
            buf = bufs[slot]
            for k in range(W):
                pltpu.make_async_copy(
                    ef_hbm.at[pl.ds(base, CH), :],
                    buf.at[pl.ds(k * CH, CH), :], rsems[slot]).wait()
            for k in range(W):
                pltpu.async_copy(buf.at[pl.ds(k * CH, CH), :],
                                 table.at[idx_v.at[g * W + k]],
                                 asems[slot], add=True)
            for k in range(W):
                pltpu.make_async_copy(buf.at[pl.ds(k * CH, CH), :],
                                      table.at[idx_v.at[k]],
                                      asems[slot]).wait()

        _pingpong(nfull, fire, consume)
        for t in range(rem):
            j = nfull * W + t
            pltpu.sync_copy(ef_hbm.at[pl.ds(base + j * CH, CH), :],
                            big1.at[pl.ds(0, CH), :])
            pltpu.sync_copy(big1.at[pl.ds(0, CH), :],
                            table.at[idx_v.at[j]], add=True)

        plsc.subcore_barrier()

        @pl.when(s < N // ZR)
        def _():
            pltpu.sync_copy(table.at[pl.ds(s * ZR, ZR), :],
                            parts_hbm.at[pl.ds(c * N + s * ZR, ZR), :])

    f = pl.kernel(
        body,
        out_type=jax.ShapeDtypeStruct((2 * N, D), jnp.float32),
        mesh=_sc_mesh(),
        scratch_types=[
            pltpu.VMEM((cpt, CH), jnp.int32),
            pltpu.VMEM((W * CH, D), jnp.float32),
            pltpu.VMEM((W * CH, D), jnp.float32),
            pltpu.VMEM_SHARED((N, D), jnp.float32),
            pltpu.SemaphoreType.DMA,
            pltpu.SemaphoreType.DMA,
            pltpu.SemaphoreType.DMA,
            pltpu.SemaphoreType.DMA,
        ],
    )
    return f(ef, idx3d, init2n)


def _layer_norm(y, g, b):
    m = jnp.mean(y, axis=-1, keepdims=True)
    v = jnp.mean((y - m) ** 2, axis=-1, keepdims=True)
    return (y - m) * lax.rsqrt(v + 1e-5) * g + b


def _edge_mlp_kernel(e_ref, gs_ref, gd_ref, w1_ref, b1_ref, w2_ref, b2_ref,
                     w3_ref, b3_ref, g_ref, beta_ref, out_ref):
    e = e_ref[...]
    x = jnp.concatenate([e, gs_ref[...], gd_ref[...]], axis=1)
    h = jnp.maximum(
        jnp.dot(x, w1_ref[...], preferred_element_type=jnp.float32)
        + b1_ref[...], 0.0)
    h = jnp.maximum(
        jnp.dot(h, w2_ref[...], preferred_element_type=jnp.float32)
        + b2_ref[...], 0.0)
    y = jnp.dot(h, w3_ref[...], preferred_element_type=jnp.float32) + b3_ref[...]
    out_ref[...] = _layer_norm(y, g_ref[...], beta_ref[...]) + e


def _edge_mlp_call(efeat, gs, gd, w1, b1, w2, b2, w3, b3, g, beta, E, D):
    BR = next(b for b in (2000, 1600, 1000, 800, 400, 200, 80, 40, 8)
              if E % b == 0)
    grid = (E // BR,)
    row = lambda i: (i, 0)
    full = lambda i: (0, 0)
    return pl.pallas_call(
        _edge_mlp_kernel,
        grid=grid,
        in_specs=[
            pl.BlockSpec((BR, D), row),
            pl.BlockSpec((BR, D), row),
            pl.BlockSpec((BR, D), row),
            pl.BlockSpec((3 * D, D), full),
            pl.BlockSpec((1, D), full),
            pl.BlockSpec((D, D), full),
            pl.BlockSpec((1, D), full),
            pl.BlockSpec((D, D), full),
            pl.BlockSpec((1, D), full),
            pl.BlockSpec((1, D), full),
            pl.BlockSpec((1, D), full),
        ],
        out_specs=pl.BlockSpec((BR, D), row),
        out_shape=jax.ShapeDtypeStruct((E, D), jnp.float32),
    )(efeat, gs, gd, w1, b1, w2, b2, w3, b3, g, beta)


def _node_mlp_kernel(p_ref, nf_ref, w1_ref, b1_ref, w2_ref, b2_ref, w3_ref,
                     b3_ref, g_ref, beta_ref, out_ref):
    nf = nf_ref[...]
    agg = p_ref[0] + p_ref[1]
    x = jnp.concatenate([agg, nf], axis=1)
    h = jnp.maximum(
        jnp.dot(x, w1_ref[...], preferred_element_type=jnp.float32)
        + b1_ref[...], 0.0)
    h = jnp.maximum(
        jnp.dot(h, w2_ref[...], preferred_element_type=jnp.float32)
        + b2_ref[...], 0.0)
    y = jnp.dot(h, w3_ref[...], preferred_element_type=jnp.float32) + b3_ref[...]
    out_ref[...] = _layer_norm(y, g_ref[...], beta_ref[...]) + nf


def _node_mlp_call(parts, nfeat, w1, b1, w2, b2, w3, b3, g, beta, N, D):
    BR = 2000
    grid = (N // BR,)
    row = lambda i: (i, 0)
    full = lambda i: (0, 0)
    parts3 = parts.reshape(2, N, D)
    return pl.pallas_call(
        _node_mlp_kernel,
        grid=grid,
        in_specs=[
            pl.BlockSpec((2, BR, D), lambda i: (0, i, 0)),
            pl.BlockSpec((BR, D), row),
            pl.BlockSpec((2 * D, D), full),
            pl.BlockSpec((1, D), full),
            pl.BlockSpec((D, D), full),
            pl.BlockSpec((1, D), full),
            pl.BlockSpec((D, D), full),
            pl.BlockSpec((1, D), full),
            pl.BlockSpec((1, D), full),
            pl.BlockSpec((1, D), full),
        ],
        out_specs=pl.BlockSpec((BR, D), row),
        out_shape=jax.ShapeDtypeStruct((N, D), jnp.float32),
    )(parts3, nfeat, w1, b1, w2, b2, w3, b3, g, beta)


def kernel(node_features, edge_features, edge_index, context_node, context_edge,
           eW1, eb1, eW2, eb2, eW3, eb3, eg, ebeta,
           nW1, nb1, nW2, nb2, nW3, nb3, ng, nbeta):
    N, D = node_features.shape
    E = edge_features.shape[0]
    L = eW1.shape[0]

    EA = (E * 2 // 5 // (CH * NW)) * (CH * NW)
    EB = E - EA
    cptA = EA // CH // NW
    cptB = EB // CH // NW

    giA = edge_index[:, :EA].reshape(2 * NW, cptA, CH)
    giB = edge_index[:, EA:].reshape(2 * NW, cptB, CH)
    diA = edge_index[1, :EA].reshape(NW, cptA, CH)
    diB = edge_index[1, EA:].reshape(NW, cptB, CH)
    zeros_2n = jnp.zeros((2 * N, D), jnp.float32)

    r = lambda b: b.reshape(1, D)

    nfeat = node_features
    efA = edge_features[:EA]
    efB = edge_features[EA:]
    for l in range(L):
        gsA, gdA = _gather_call(nfeat, giA, EA, D)
        gsB, gdB = _gather_call(nfeat, giB, EB, D)
        efA = _edge_mlp_call(efA, gsA, gdA, eW1[l], r(eb1[l]), eW2[l],
                             r(eb2[l]), eW3[l], r(eb3[l]), r(eg[l]),
                             r(ebeta[l]), EA, D)
        efB = _edge_mlp_call(efB, gsB, gdB, eW1[l], r(eb1[l]), eW2[l],
                             r(eb2[l]), eW3[l], r(eb3[l]), r(eg[l]),
                             r(ebeta[l]), EB, D)
        partsA = _scatter_call(efA, diA, zeros_2n, N, D)
        parts = _scatter_call(efB, diB, partsA, N, D)
        nfeat = _node_mlp_call(parts, nfeat, nW1[l], r(nb1[l]), nW2[l],
                               r(nb2[l]), nW3[l], r(nb3[l]), r(ng[l]),
                               r(nbeta[l]), N, D)
    return nfeat
